# Initial kernel scaffold; baseline (speedup 1.0000x reference)
import jax
import jax.numpy as jnp
from jax.experimental import pallas as pl


def kernel(x, nei, nei_mask, kernel_tangents, W, b):
    raise NotImplementedError("write your pallas kernel here")


# speedup vs baseline: 3.3511x; 3.3511x over previous
import functools

import jax
import jax.numpy as jnp
from jax import lax
from jax.experimental import pallas as pl
from jax.experimental.pallas import tpu as pltpu
from jax.experimental.pallas import tpu_sc as plsc

KP_EXTENT = 0.66
MIN_NORM = 1e-15
MAXNORM = 1.0 - 1e-5

D = 128
K = 4
NEI = 16

B1 = 1000
B3 = 200
GATHER_WIN = 128


def _artanh(z):
    z = jnp.clip(z, -1.0 + 1e-7, 1.0 - 1e-7)
    return 0.5 * jnp.log((1.0 + z) / (1.0 - z))


def _proj(z, n2=None):
    if n2 is None:
        n2 = jnp.sum(z * z, -1, keepdims=True)
    norm = jnp.maximum(jnp.sqrt(n2), MIN_NORM)
    return jnp.where(norm > MAXNORM, z * (MAXNORM / norm), z)


def _pack2(a, b):
    au = lax.convert_element_type(
        lax.bitcast_convert_type(a.astype(jnp.bfloat16), jnp.uint16),
        jnp.uint32)
    bu = lax.convert_element_type(
        lax.bitcast_convert_type(b.astype(jnp.bfloat16), jnp.uint16),
        jnp.uint32)
    return lax.bitcast_convert_type(au | (bu << 16), jnp.float32)


def _unpack2(w):
    u = lax.bitcast_convert_type(w, jnp.uint32)
    a = lax.bitcast_convert_type(u << 16, jnp.float32)
    b = lax.bitcast_convert_type(u & jnp.uint32(0xFFFF0000), jnp.float32)
    return a, b


def _stage1_body(x_ref, w_ref, b_ref, g01_ref, g23_ref):
    xb = x_ref[...]
    wf = w_ref[...]
    bf = b_ref[...]

    x2 = jnp.sum(xb * xb, -1, keepdims=True)
    pn = jnp.maximum(jnp.sqrt(x2), MIN_NORM)
    t = _artanh(pn) * xb / pn

    bn = jnp.maximum(
        jnp.sqrt(jnp.sum(bf * bf, -1, keepdims=True)), MIN_NORM)
    hb = _proj(jnp.tanh(bn) * bf / bn)
    hb2 = jnp.sum(hb * hb, -1, keepdims=True)

    packed = []
    for k in range(K):
        u = jnp.dot(t, wf[k].T, preferred_element_type=jnp.float32)
        un2 = jnp.sum(u * u, -1, keepdims=True)
        un = jnp.maximum(jnp.sqrt(un2), MIN_NORM)
        feat = _proj(jnp.tanh(un) * u / un)
        f2 = jnp.sum(feat * feat, -1, keepdims=True)
        fy = jnp.sum(feat * hb[k][None, :], -1, keepdims=True)
        y2 = hb2[k][None, :]
        num = (1.0 + 2.0 * fy + y2) * feat + (1.0 - f2) * hb[k][None, :]
        den = 1.0 + 2.0 * fy + f2 * y2
        m = _proj(num / jnp.maximum(den, MIN_NORM))
        m2 = jnp.sum(m * m, -1, keepdims=True)
        kf = 2.0 * m / (1.0 + m2)
        k2 = jnp.sum(kf * kf, -1, keepdims=True)
        gam = lax.rsqrt(jnp.maximum(1.0 - k2, MIN_NORM))
        gg = gam * kf
        packed.append(_pack2(gg[:, :64], gg[:, 64:]))
    g01_ref[...] = jnp.concatenate(packed[0:2], axis=-1)
    g23_ref[...] = jnp.concatenate(packed[2:4], axis=-1)


def _build_tables(x, W, b):
    n = x.shape[0]
    out = jax.ShapeDtypeStruct((n, D), jnp.float32)
    return pl.pallas_call(
        _stage1_body,
        grid=(n // B1,),
        in_specs=[
            pl.BlockSpec((B1, D), lambda i: (i, 0)),
            pl.BlockSpec((K, D, D), lambda i: (0, 0, 0)),
            pl.BlockSpec((K, D), lambda i: (0, 0)),
        ],
        out_specs=[pl.BlockSpec((B1, D), lambda i: (i, 0))] * 2,
        out_shape=[out, out],
    )(x, W, b)


def _sc_gather(x, g01, g23, idx_flat):
    num_idx = idx_flat.shape[0]
    idx2 = idx_flat.reshape(1, num_idx)
    mesh = plsc.VectorSubcoreMesh(
        core_axis_name="core", subcore_axis_name="subcore")
    out = jax.ShapeDtypeStruct((num_idx, D), jnp.float32)

    @functools.partial(pl.kernel, out_type=[out, out, out], mesh=mesh)
    def gather_kernel(x_hbm, a_hbm, b_hbm, i_hbm, ox_hbm, oa_hbm, ob_hbm):
        def body(i_vmem, ox_v, oa_v, ob_v):
            pltpu.sync_copy(x_hbm.at[i_vmem.at[0]], ox_v)
            pltpu.sync_copy(a_hbm.at[i_vmem.at[0]], oa_v)
            pltpu.sync_copy(b_hbm.at[i_vmem.at[0]], ob_v)

        pltpu.emit_pipeline(
            body,
            grid=(num_idx // GATHER_WIN,),
            in_specs=[pl.BlockSpec((1, GATHER_WIN),
                                   index_map=lambda i: (0, i))],
            out_specs=[pl.BlockSpec((GATHER_WIN, D),
                                    index_map=lambda i: (i, 0))] * 3,
            core_axis_name=("core", "subcore"),
            dimension_semantics=(pltpu.PARALLEL,),
        )(i_hbm, ox_hbm, oa_hbm, ob_hbm)

    return gather_kernel(x, g01, g23, idx2)


def _stage3_body(x_ref, kt_ref, mask_ref, xg_ref, g01_ref, g23_ref, o_ref):
    xb = x_ref[...]
    kt = kt_ref[...]
    mask = mask_ref[...]

    xn = xg_ref[...].reshape(B3, NEI, D)
    y2 = jnp.sum(xn * xn, -1)

    x2 = jnp.sum(xb * xb, -1, keepdims=True)
    one_m_x2 = 1.0 - x2

    xkernels = []
    for k in range(1, K):
        tmp = one_m_x2 * kt[k][None, :]
        t2 = jnp.sum(tmp * tmp, -1, keepdims=True)
        tmp = _proj(tmp, t2)
        t2 = jnp.sum(tmp * tmp, -1, keepdims=True)
        tn = jnp.maximum(jnp.sqrt(t2), MIN_NORM)
        lam = 2.0 / jnp.maximum(one_m_x2, MIN_NORM)
        second = jnp.tanh(lam * tn / 2.0) * tmp / tn
        s2 = jnp.sum(second * second, -1, keepdims=True)
        xs = jnp.sum(xb * second, -1, keepdims=True)
        num = (1.0 + 2.0 * xs + s2) * xb + (1.0 - x2) * second
        den = 1.0 + 2.0 * xs + x2 * s2
        xkernels.append(_proj(num / jnp.maximum(den, MIN_NORM)))
    xkernels.append(xb)

    g01 = g01_ref[...]
    g23 = g23_ref[...]
    ggs = []
    for k in range(K):
        src = g01 if k < 2 else g23
        lo, hi = _unpack2(src[:, (k % 2) * 64:(k % 2) * 64 + 64])
        ggs.append(jnp.concatenate([lo, hi], axis=-1).reshape(B3, NEI, D))

    num2 = jnp.zeros((B3, D), jnp.float32)
    den2 = jnp.zeros((B3, 1), jnp.float32)
    for k in range(K):
        a = xkernels[k]
        a2 = jnp.sum(a * a, -1, keepdims=True)
        ay = jnp.sum(a[:, None, :] * xn, -1)
        A = 1.0 - 2.0 * ay + y2
        Bc = 1.0 - a2
        nn2 = A * A * a2 + Bc * Bc * y2 - 2.0 * A * Bc * ay
        dd = jnp.maximum(1.0 - 2.0 * ay + a2 * y2, MIN_NORM)
        un = jnp.maximum(jnp.sqrt(jnp.maximum(nn2, 0.0)) / dd, MIN_NORM)
        dis = 2.0 * _artanh(un)
        w = jnp.maximum(0.0, 1.0 - dis / KP_EXTENT) * mask
        gg = ggs[k]
        gam = jnp.sqrt(1.0 + jnp.sum(gg * gg, -1))
        num_k = jnp.sum(w[:, :, None] * gg, axis=1)
        den_k = jnp.maximum(
            jnp.sum(w * gam, axis=1, keepdims=True), MIN_NORM)
        mid = num_k / den_k
        m2 = jnp.sum(mid * mid, -1, keepdims=True)
        g2 = lax.rsqrt(jnp.maximum(1.0 - m2, MIN_NORM))
        num2 = num2 + g2 * mid
        den2 = den2 + g2
    midk = num2 / jnp.maximum(den2, MIN_NORM)
    mk2 = jnp.sum(midk * midk, -1, keepdims=True)
    p = midk / (1.0 + jnp.sqrt(jnp.maximum(1.0 - mk2, MIN_NORM)))
    o_ref[...] = _proj(p)


def _aggregate(x, kt, nei_mask, xg, g01, g23):
    n = x.shape[0]
    gspec = pl.BlockSpec((B3 * NEI, D), lambda i: (i, 0))
    return pl.pallas_call(
        _stage3_body,
        grid=(n // B3,),
        in_specs=[
            pl.BlockSpec((B3, D), lambda i: (i, 0)),
            pl.BlockSpec((K, D), lambda i: (0, 0)),
            pl.BlockSpec((B3, NEI), lambda i: (i, 0)),
            gspec, gspec, gspec,
        ],
        out_specs=pl.BlockSpec((B3, D), lambda i: (i, 0)),
        out_shape=jax.ShapeDtypeStruct((n, D), jnp.float32),
    )(x, kt, nei_mask, xg, g01, g23)


def kernel(x, nei, nei_mask, kernel_tangents, W, b):
    n = x.shape[0]
    g01, g23 = _build_tables(x, W, b)
    xg, ga, gb = _sc_gather(x, g01, g23, nei.reshape(n * NEI))
    return _aggregate(x, kernel_tangents, nei_mask, xg, ga, gb)
